# baseline (speedup 1.0000x reference)
import functools

import jax
import jax.numpy as jnp
from jax.experimental import pallas as pl
from jax.experimental.pallas import tpu as pltpu


def _round_up(x: int, m: int) -> int:
    return ((x + m - 1) // m) * m


def _make_transform_kernel(mf_dim: int):
    def _transform(tab_ref, w_ref, b_ref, s_ref, out_ref):
        f32 = jnp.float32
        g = tab_ref[:, :mf_dim] * s_ref[...]
        m = jnp.dot(tab_ref[:, mf_dim:], w_ref[...],
                    preferred_element_type=f32) + b_ref[...]
        out_ref[...] = jnp.concatenate([g, m], axis=1)
    return _transform


def _make_main_kernel(tile_b: int, mf_dim: int):
    def _main(uidx_ref, iidx_ref,
              tu_ref, ti_ref,
              w2_ref, b2_ref, w3_ref, b3_ref,
              wfm_ref, bf_ref,
              out_ref,
              au_ref, ai_ref):
        f32 = jnp.float32
        for r in range(tile_b):
            au_ref[r, :] = tu_ref[uidx_ref[0, 0, r], 0]
            ai_ref[r, :] = ti_ref[iidx_ref[0, 0, r], 0]
        a = au_ref[...]
        b = ai_ref[...]
        h = jnp.maximum(a[:, mf_dim:] + b[:, mf_dim:], 0.0)
        h = jnp.maximum(
            jnp.dot(h, w2_ref[...], preferred_element_type=f32) + b2_ref[...], 0.0)
        h = jnp.maximum(
            jnp.dot(h, w3_ref[...], preferred_element_type=f32) + b3_ref[...], 0.0)
        s8 = a[:, :mf_dim] * b[:, :mf_dim] + h * wfm_ref[...]
        score = jnp.sum(s8, axis=1, keepdims=True) + bf_ref[...]
        out_ref[...] = jax.nn.sigmoid(score)
    return _main


def _transform_table(tab, w, b_row, s_row, *, row_tile: int):
    n, width = tab.shape
    mf_dim = width - w.shape[0]
    n_pad = _round_up(n, row_tile)
    tab_p = jnp.pad(tab, ((0, n_pad - n), (0, 0)))
    grid = n_pad // row_tile
    out = pl.pallas_call(
        _make_transform_kernel(mf_dim),
        out_shape=jax.ShapeDtypeStruct((n_pad, width), jnp.float32),
        grid=(grid,),
        in_specs=[
            pl.BlockSpec((row_tile, width), lambda i: (i, 0)),
            pl.BlockSpec(w.shape, lambda i: (0, 0)),
            pl.BlockSpec(b_row.shape, lambda i: (0, 0)),
            pl.BlockSpec(s_row.shape, lambda i: (0, 0)),
        ],
        out_specs=pl.BlockSpec((row_tile, width), lambda i: (i, 0)),
        compiler_params=pltpu.CompilerParams(
            dimension_semantics=("parallel",)),
    )(tab_p, w, b_row, s_row)
    return out


@functools.partial(jax.jit, static_argnames=("tile_b",))
def _forward(user_idx, item_idx, user_emb, item_emb,
             w1, b1, w2, b2, w3, b3, wf, bf, *, tile_b: int = 256):
    B = int(user_idx.shape[0])
    U, W = user_emb.shape
    half = w1.shape[0] // 2
    mf_dim = W - half
    l3 = w3.shape[1]

    wf_g = wf[:mf_dim, :].T
    ones_g = jnp.ones((1, mf_dim), jnp.float32)
    zero_b = jnp.zeros_like(b1)
    tu = _transform_table(user_emb, w1[:half, :], zero_b, wf_g, row_tile=1024)
    ti = _transform_table(item_emb, w1[half:, :], b1, ones_g, row_tile=1024)
    tu3 = tu.reshape(tu.shape[0], 1, W)
    ti3 = ti.reshape(ti.shape[0], 1, W)

    wfm_row = wf[mf_dim:, :].T

    b_pad = _round_up(B, 2 * tile_b)
    pad = b_pad - B
    uidx = jnp.pad(user_idx.astype(jnp.int32), (0, pad)).reshape(-1, 1, tile_b)
    iidx = jnp.pad(item_idx.astype(jnp.int32), (0, pad)).reshape(-1, 1, tile_b)
    num_tiles = b_pad // tile_b

    idx_spec = pl.BlockSpec((1, 1, tile_b), lambda i: (i, 0, 0),
                            memory_space=pltpu.SMEM)

    def _whole(a):
        return pl.BlockSpec(a.shape, lambda c, i: (0,) * a.ndim)

    half_tiles = num_tiles // 2

    def _idx_map(c, i):
        return (c * half_tiles + i, 0, 0)

    out = pl.pallas_call(
        _make_main_kernel(tile_b, mf_dim),
        out_shape=jax.ShapeDtypeStruct((b_pad, 1), jnp.float32),
        grid=(2, half_tiles),
        in_specs=[pl.BlockSpec((1, 1, tile_b), _idx_map, memory_space=pltpu.SMEM),
                  pl.BlockSpec((1, 1, tile_b), _idx_map, memory_space=pltpu.SMEM),
                  _whole(tu3), _whole(ti3),
                  _whole(w2), _whole(b2), _whole(w3), _whole(b3),
                  _whole(wfm_row), _whole(bf)],
        out_specs=pl.BlockSpec((tile_b, 1),
                               lambda c, i: (c * half_tiles + i, 0)),
        scratch_shapes=[pltpu.VMEM((tile_b, W), jnp.float32),
                        pltpu.VMEM((tile_b, W), jnp.float32)],
        compiler_params=pltpu.CompilerParams(
            dimension_semantics=("parallel", "arbitrary"),
            vmem_limit_bytes=64 * 1024 * 1024,
        ),
    )(uidx, iidx, tu3, ti3, w2, b2, w3, b3, wfm_row, bf)
    return out[:B]


def kernel(user_idx, item_idx, user_emb, item_emb, w1, b1, w2, b2, w3, b3, wf, bf):
    return _forward(user_idx, item_idx, user_emb, item_emb,
                    w1, b1, w2, b2, w3, b3, wf, bf)


# speedup vs baseline: 1.4763x; 1.4763x over previous
import functools

import jax
import jax.numpy as jnp
from jax.experimental import pallas as pl
from jax.experimental.pallas import tpu as pltpu


def _round_up(x: int, m: int) -> int:
    return ((x + m - 1) // m) * m


def _make_transform_kernel(mf_dim: int):
    def _transform(tab_ref, w_ref, b_ref, s_ref, out_ref):
        f32 = jnp.float32
        g = tab_ref[:, :mf_dim] * s_ref[...]
        m = jnp.dot(tab_ref[:, mf_dim:], w_ref[...],
                    preferred_element_type=f32) + b_ref[...]
        out_ref[...] = jnp.concatenate([g, m], axis=1)
    return _transform


def _make_main_kernel(tile_b: int, chunk: int):
    n_chunks = tile_b // chunk

    def _main(uidx_ref, iidx_ref,
              tu_ref, ti_ref,
              w2p_ref, b2_ref,
              w3_ref, b3_ref,
              selg_ref, wfm_ref,
              bf_ref,
              out_ref,
              *scratch):
        f32 = jnp.float32
        for c in range(n_chunks):
            au_ref = scratch[2 * c]
            ai_ref = scratch[2 * c + 1]
            base = c * chunk
            for r in range(chunk):
                au_ref[r, :] = tu_ref[uidx_ref[0, 0, base + r], 0]
                ai_ref[r, :] = ti_ref[iidx_ref[0, 0, base + r], 0]
            a = au_ref[...]
            b = ai_ref[...]
            h = jnp.maximum(a + b, 0.0)
            h = jnp.maximum(
                jnp.dot(h, w2p_ref[...], preferred_element_type=f32)
                + b2_ref[...], 0.0)
            h = jnp.maximum(
                jnp.dot(h, w3_ref[...], preferred_element_type=f32)
                + b3_ref[...], 0.0)
            g = a * b
            score = (jnp.dot(g, selg_ref[...], preferred_element_type=f32)
                     + jnp.dot(h, wfm_ref[...], preferred_element_type=f32)
                     + bf_ref[...])
            out_ref[pl.ds(base, chunk), :] = jax.nn.sigmoid(score)
    return _main


def _transform_table(tab, w, b_row, s_row, *, row_tile: int):
    n, width = tab.shape
    mf_dim = width - w.shape[0]
    n_pad = _round_up(n, row_tile)
    tab_p = jnp.pad(tab, ((0, n_pad - n), (0, 0)))
    grid = n_pad // row_tile
    out = pl.pallas_call(
        _make_transform_kernel(mf_dim),
        out_shape=jax.ShapeDtypeStruct((n_pad, width), jnp.float32),
        grid=(grid,),
        in_specs=[
            pl.BlockSpec((row_tile, width), lambda i: (i, 0)),
            pl.BlockSpec(w.shape, lambda i: (0, 0)),
            pl.BlockSpec(b_row.shape, lambda i: (0, 0)),
            pl.BlockSpec(s_row.shape, lambda i: (0, 0)),
        ],
        out_specs=pl.BlockSpec((row_tile, width), lambda i: (i, 0)),
        compiler_params=pltpu.CompilerParams(
            dimension_semantics=("parallel",)),
    )(tab_p, w, b_row, s_row)
    return out


@functools.partial(jax.jit, static_argnames=("tile_b", "chunk"))
def _forward(user_idx, item_idx, user_emb, item_emb,
             w1, b1, w2, b2, w3, b3, wf, bf, *,
             tile_b: int = 1024, chunk: int = 256):
    B = int(user_idx.shape[0])
    U, W = user_emb.shape
    half = w1.shape[0] // 2
    mf_dim = W - half
    l2 = w2.shape[1]

    wf_g = wf[:mf_dim, :].T
    ones_g = jnp.ones((1, mf_dim), jnp.float32)
    zero_b = jnp.zeros_like(b1)
    tu = _transform_table(user_emb, w1[:half, :], zero_b, wf_g, row_tile=1024)
    ti = _transform_table(item_emb, w1[half:, :], b1, ones_g, row_tile=1024)
    tu3 = tu.reshape(tu.shape[0], 1, W)
    ti3 = ti.reshape(ti.shape[0], 1, W)

    w2p = jnp.concatenate([jnp.zeros((mf_dim, l2), jnp.float32), w2], axis=0)
    selg = jnp.concatenate([jnp.ones((mf_dim, 1), jnp.float32),
                            jnp.zeros((half, 1), jnp.float32)], axis=0)
    wfm_col = wf[mf_dim:, :]

    b_pad = _round_up(B, tile_b)
    pad = b_pad - B
    uidx = jnp.pad(user_idx.astype(jnp.int32), (0, pad)).reshape(-1, 1, tile_b)
    iidx = jnp.pad(item_idx.astype(jnp.int32), (0, pad)).reshape(-1, 1, tile_b)
    num_tiles = b_pad // tile_b

    idx_spec = pl.BlockSpec((1, 1, tile_b), lambda i: (i, 0, 0),
                            memory_space=pltpu.SMEM)

    def _whole(a):
        return pl.BlockSpec(a.shape, lambda i: (0,) * a.ndim)

    out = pl.pallas_call(
        _make_main_kernel(tile_b, chunk),
        out_shape=jax.ShapeDtypeStruct((b_pad, 1), jnp.float32),
        grid=(num_tiles,),
        in_specs=[idx_spec, idx_spec,
                  _whole(tu3), _whole(ti3),
                  _whole(w2p), _whole(b2), _whole(w3), _whole(b3),
                  _whole(selg), _whole(wfm_col), _whole(bf)],
        out_specs=pl.BlockSpec((tile_b, 1), lambda i: (i, 0)),
        scratch_shapes=[pltpu.VMEM((chunk, W), jnp.float32)
                        for _ in range(2 * (tile_b // chunk))],
        compiler_params=pltpu.CompilerParams(
            dimension_semantics=("parallel",),
            vmem_limit_bytes=64 * 1024 * 1024,
        ),
    )(uidx, iidx, tu3, ti3, w2p, b2, w3, b3, selg, wfm_col, bf)
    return out[:B]


def kernel(user_idx, item_idx, user_emb, item_emb, w1, b1, w2, b2, w3, b3, wf, bf):
    return _forward(user_idx, item_idx, user_emb, item_emb,
                    w1, b1, w2, b2, w3, b3, wf, bf)
